# baseline (speedup 1.0000x reference)
import functools

import jax
import jax.numpy as jnp
from jax import lax
from jax.experimental import pallas as pl
from jax.experimental.pallas import tpu as pltpu
from jax.experimental.pallas import tpu_sc as plsc

N = 10000
D = 128
D_OUT = 64
E = 320000

NC = 2
NS = 16
NW = NC * NS
CHUNK = 128
CHUNKS = 80
GROUP = 40
E_PAD = NW * CHUNKS * CHUNK
N_PAD = 10240
ROWS_PER_TILE = N_PAD // NS


def _sc_scatter_build():
    mesh = plsc.VectorSubcoreMesh(core_axis_name="c", subcore_axis_name="s")

    @functools.partial(
        pl.kernel,
        mesh=mesh,
        out_type=jax.ShapeDtypeStruct((NC, N_PAD, D), jnp.float32),
        scratch_types=[
            pltpu.VMEM((GROUP, CHUNK), jnp.int32),
            pltpu.VMEM((GROUP, CHUNK), jnp.int32),
            pltpu.VMEM((CHUNK, D), jnp.float32),
            pltpu.VMEM((CHUNK, D), jnp.float32),
            pltpu.VMEM_SHARED((N_PAD, D), jnp.float32),
            pltpu.SemaphoreType.DMA,
            pltpu.SemaphoreType.DMA,
        ],
    )
    def sc_scatter(src_hbm, dst_hbm, x_hbm, out_hbm,
                   src_v, dst_v, rows_0, rows_1, acc_sh, sem_0, sem_1):
        c = lax.axis_index("c")
        s = lax.axis_index("s")
        w = c * NS + s

        pltpu.sync_copy(x_hbm.at[pl.ds(s * ROWS_PER_TILE, ROWS_PER_TILE)],
                        acc_sh.at[pl.ds(s * ROWS_PER_TILE, ROWS_PER_TILE)])
        plsc.subcore_barrier()

        rows = (rows_0, rows_1)
        sems = (sem_0, sem_1)
        for h in range(CHUNKS // GROUP):
            pltpu.sync_copy(src_hbm.at[w, pl.ds(h * GROUP, GROUP)], src_v)
            pltpu.sync_copy(dst_hbm.at[w, pl.ds(h * GROUP, GROUP)], dst_v)

            for b in range(2):
                pltpu.async_copy(x_hbm.at[src_v.at[b]], rows[b], sems[b])

            def body2(jj, carry):
                j0 = jj * 2
                for b in range(2):
                    j = j0 + b
                    pltpu.make_async_copy(x_hbm.at[src_v.at[j]], rows[b],
                                          sems[b]).wait()
                    pltpu.sync_copy(rows[b], acc_sh.at[dst_v.at[j]], add=True)

                    @pl.when(j + 2 < GROUP)
                    def _():
                        pltpu.async_copy(x_hbm.at[src_v.at[j + 2]], rows[b],
                                         sems[b])

                return carry

            lax.fori_loop(0, GROUP // 2, body2, 0)
        plsc.subcore_barrier()

        pltpu.sync_copy(acc_sh.at[pl.ds(s * ROWS_PER_TILE, ROWS_PER_TILE)],
                        out_hbm.at[c, pl.ds(s * ROWS_PER_TILE, ROWS_PER_TILE)])

    return sc_scatter


_sc_scatter = _sc_scatter_build()


def _mlp_mid_body(x_ref, p_ref, wa_ref, ba_ref, wb_ref, bb_ref, o_ref):
    t = p_ref[0] + p_ref[1] - x_ref[...]
    u = jnp.maximum(
        jnp.dot(t, wa_ref[...], preferred_element_type=jnp.float32)
        + ba_ref[...], 0.0)
    v = jnp.dot(u, wb_ref[...], preferred_element_type=jnp.float32) + bb_ref[...]
    o_ref[...] = jnp.maximum(v, 0.0)


def _mlp_last_body(x_ref, p_ref, wa_ref, ba_ref, wb_ref, bb_ref,
                   w3_ref, b3_ref, o_ref):
    t = p_ref[0] + p_ref[1] - x_ref[...]
    u = jnp.maximum(
        jnp.dot(t, wa_ref[...], preferred_element_type=jnp.float32)
        + ba_ref[...], 0.0)
    v = jnp.dot(u, wb_ref[...], preferred_element_type=jnp.float32) + bb_ref[...]
    h = jnp.maximum(v, 0.0)
    o_ref[...] = (jnp.dot(h, w3_ref[...], preferred_element_type=jnp.float32)
                  + b3_ref[...])


_RB = 1024


def _tc_mlp_mid(x, p, wa, ba, wb, bb):
    grid = (N_PAD // _RB,)
    return pl.pallas_call(
        _mlp_mid_body,
        grid=grid,
        in_specs=[
            pl.BlockSpec((_RB, D), lambda i: (i, 0)),
            pl.BlockSpec((NC, _RB, D), lambda i: (0, i, 0)),
            pl.BlockSpec((D, D), lambda i: (0, 0)),
            pl.BlockSpec((1, D), lambda i: (0, 0)),
            pl.BlockSpec((D, D), lambda i: (0, 0)),
            pl.BlockSpec((1, D), lambda i: (0, 0)),
        ],
        out_specs=pl.BlockSpec((_RB, D), lambda i: (i, 0)),
        out_shape=jax.ShapeDtypeStruct((N_PAD, D), jnp.float32),
    )(x, p, wa, ba, wb, bb)


def _tc_mlp_last(x, p, wa, ba, wb, bb, w3, b3):
    grid = (N // _RB + 1,)
    return pl.pallas_call(
        _mlp_last_body,
        grid=grid,
        in_specs=[
            pl.BlockSpec((_RB, D), lambda i: (i, 0)),
            pl.BlockSpec((NC, _RB, D), lambda i: (0, i, 0)),
            pl.BlockSpec((D, D), lambda i: (0, 0)),
            pl.BlockSpec((1, D), lambda i: (0, 0)),
            pl.BlockSpec((D, D), lambda i: (0, 0)),
            pl.BlockSpec((1, D), lambda i: (0, 0)),
            pl.BlockSpec((D, D_OUT), lambda i: (0, 0)),
            pl.BlockSpec((1, D_OUT), lambda i: (0, 0)),
        ],
        out_specs=pl.BlockSpec((_RB, D_OUT), lambda i: (i, 0)),
        out_shape=jax.ShapeDtypeStruct((N, D_OUT), jnp.float32),
    )(x, p, wa, ba, wb, bb, w3, b3)


def kernel(x, edge_index, W1a, b1a, W1b, b1b, W2a, b2a, W2b, b2b, W3, b3):
    src = edge_index[0].astype(jnp.int32)
    dst = edge_index[1].astype(jnp.int32)
    pad = E_PAD - E
    src_p = jnp.concatenate([src, jnp.zeros((pad,), jnp.int32)])
    dst_p = jnp.concatenate([dst, jnp.full((pad,), N, jnp.int32)])
    src_r = src_p.reshape(NW, CHUNKS, CHUNK)
    dst_r = dst_p.reshape(NW, CHUNKS, CHUNK)

    x_pad = jnp.concatenate([x, jnp.zeros((N_PAD - N, D), jnp.float32)])

    b1a2 = b1a.reshape(1, D)
    b1b2 = b1b.reshape(1, D)
    b2a2 = b2a.reshape(1, D)
    b2b2 = b2b.reshape(1, D)
    b32 = b3.reshape(1, D_OUT)

    p1 = _sc_scatter(src_r, dst_r, x_pad)
    h1 = _tc_mlp_mid(x_pad, p1, W1a, b1a2, W1b, b1b2)
    p2 = _sc_scatter(src_r, dst_r, h1)
    out = _tc_mlp_last(h1, p2, W2a, b2a2, W2b, b2b2, W3, b32)
    return out


# speedup vs baseline: 2.1014x; 2.1014x over previous
import functools

import jax
import jax.numpy as jnp
from jax import lax
from jax.experimental import pallas as pl
from jax.experimental.pallas import tpu as pltpu
from jax.experimental.pallas import tpu_sc as plsc

N = 10000
D = 128
D_OUT = 64
E = 320000

NC = 2
NS = 16
NW = NC * NS
CHUNK = 125
CHUNKS = 80
GROUP = 40
N_PAD = 10240
ROWS_PER_TILE = N_PAD // NS


def _sc_scatter_build():
    mesh = plsc.VectorSubcoreMesh(core_axis_name="c", subcore_axis_name="s")

    @functools.partial(
        pl.kernel,
        mesh=mesh,
        out_type=jax.ShapeDtypeStruct((NC, N_PAD, D), jnp.float32),
        scratch_types=[
            pltpu.VMEM((GROUP, CHUNK), jnp.int32),
            pltpu.VMEM((GROUP, CHUNK), jnp.int32),
            pltpu.VMEM((CHUNK, D), jnp.float32),
            pltpu.VMEM_SHARED((N_PAD, D), jnp.float32),
            pltpu.SemaphoreType.DMA,
        ],
    )
    def sc_scatter(src_hbm, dst_hbm, x_hbm, out_hbm,
                   src_v, dst_v, rows_0, acc_sh, sem_0):
        c = lax.axis_index("c")
        s = lax.axis_index("s")
        w = c * NS + s

        pltpu.sync_copy(x_hbm.at[pl.ds(s * ROWS_PER_TILE, ROWS_PER_TILE)],
                        acc_sh.at[pl.ds(s * ROWS_PER_TILE, ROWS_PER_TILE)])
        plsc.subcore_barrier()

        for h in range(CHUNKS // GROUP):
            pltpu.sync_copy(src_hbm.at[w, pl.ds(h * GROUP, GROUP)], src_v)
            pltpu.sync_copy(dst_hbm.at[w, pl.ds(h * GROUP, GROUP)], dst_v)

            def body(j, carry):
                pltpu.async_copy(x_hbm.at[src_v.at[j]], rows_0, sem_0).wait()
                pltpu.sync_copy(rows_0, acc_sh.at[dst_v.at[j]], add=True)
                return carry

            lax.fori_loop(0, GROUP, body, 0)
        plsc.subcore_barrier()

        pltpu.sync_copy(acc_sh.at[pl.ds(s * ROWS_PER_TILE, ROWS_PER_TILE)],
                        out_hbm.at[c, pl.ds(s * ROWS_PER_TILE, ROWS_PER_TILE)])

    return sc_scatter


_sc_scatter = _sc_scatter_build()


def _mlp_mid_body(x_ref, p_ref, wa_ref, ba_ref, wb_ref, bb_ref, o_ref):
    t = p_ref[0] + p_ref[1] - x_ref[...]
    u = jnp.maximum(
        jnp.dot(t, wa_ref[...], preferred_element_type=jnp.float32)
        + ba_ref[...], 0.0)
    v = jnp.dot(u, wb_ref[...], preferred_element_type=jnp.float32) + bb_ref[...]
    o_ref[...] = jnp.maximum(v, 0.0)


def _mlp_last_body(x_ref, p_ref, wa_ref, ba_ref, wb_ref, bb_ref,
                   w3_ref, b3_ref, o_ref):
    t = p_ref[0] + p_ref[1] - x_ref[...]
    u = jnp.maximum(
        jnp.dot(t, wa_ref[...], preferred_element_type=jnp.float32)
        + ba_ref[...], 0.0)
    v = jnp.dot(u, wb_ref[...], preferred_element_type=jnp.float32) + bb_ref[...]
    h = jnp.maximum(v, 0.0)
    o_ref[...] = (jnp.dot(h, w3_ref[...], preferred_element_type=jnp.float32)
                  + b3_ref[...])


_RB = 1024


def _tc_mlp_mid(x, p, wa, ba, wb, bb):
    grid = (N_PAD // _RB,)
    return pl.pallas_call(
        _mlp_mid_body,
        grid=grid,
        in_specs=[
            pl.BlockSpec((_RB, D), lambda i: (i, 0)),
            pl.BlockSpec((NC, _RB, D), lambda i: (0, i, 0)),
            pl.BlockSpec((D, D), lambda i: (0, 0)),
            pl.BlockSpec((1, D), lambda i: (0, 0)),
            pl.BlockSpec((D, D), lambda i: (0, 0)),
            pl.BlockSpec((1, D), lambda i: (0, 0)),
        ],
        out_specs=pl.BlockSpec((_RB, D), lambda i: (i, 0)),
        out_shape=jax.ShapeDtypeStruct((N_PAD, D), jnp.float32),
    )(x, p, wa, ba, wb, bb)


def _tc_mlp_last(x, p, wa, ba, wb, bb, w3, b3):
    grid = (N // _RB + 1,)
    return pl.pallas_call(
        _mlp_last_body,
        grid=grid,
        in_specs=[
            pl.BlockSpec((_RB, D), lambda i: (i, 0)),
            pl.BlockSpec((NC, _RB, D), lambda i: (0, i, 0)),
            pl.BlockSpec((D, D), lambda i: (0, 0)),
            pl.BlockSpec((1, D), lambda i: (0, 0)),
            pl.BlockSpec((D, D), lambda i: (0, 0)),
            pl.BlockSpec((1, D), lambda i: (0, 0)),
            pl.BlockSpec((D, D_OUT), lambda i: (0, 0)),
            pl.BlockSpec((1, D_OUT), lambda i: (0, 0)),
        ],
        out_specs=pl.BlockSpec((_RB, D_OUT), lambda i: (i, 0)),
        out_shape=jax.ShapeDtypeStruct((N, D_OUT), jnp.float32),
    )(x, p, wa, ba, wb, bb, w3, b3)


def kernel(x, edge_index, W1a, b1a, W1b, b1b, W2a, b2a, W2b, b2b, W3, b3):
    src = edge_index[0].astype(jnp.int32)
    dst = edge_index[1].astype(jnp.int32)
    src_r = src.reshape(NW, CHUNKS, CHUNK)
    dst_r = dst.reshape(NW, CHUNKS, CHUNK)

    x_pad = jnp.concatenate([x, jnp.zeros((N_PAD - N, D), jnp.float32)])

    b1a2 = b1a.reshape(1, D)
    b1b2 = b1b.reshape(1, D)
    b2a2 = b2a.reshape(1, D)
    b2b2 = b2b.reshape(1, D)
    b32 = b3.reshape(1, D_OUT)

    p1 = _sc_scatter(src_r, dst_r, x_pad)
    h1 = _tc_mlp_mid(x_pad, p1, W1a, b1a2, W1b, b1b2)
    p2 = _sc_scatter(src_r, dst_r, h1)
    out = _tc_mlp_last(h1, p2, W2a, b2a2, W2b, b2b2, W3, b32)
    return out


# speedup vs baseline: 3.0730x; 1.4623x over previous
import functools

import jax
import jax.numpy as jnp
from jax import lax
from jax.experimental import pallas as pl
from jax.experimental.pallas import tpu as pltpu
from jax.experimental.pallas import tpu_sc as plsc

N = 10000
D = 128
D_OUT = 64
E = 320000

NC = 2
NS = 16
NW = NC * NS
CHUNK = 125
CHUNKS = 80
GROUP = 40
N_PAD = 10240
ROWS_PER_TILE = N_PAD // NS


def _sc_scatter_build():
    mesh = plsc.VectorSubcoreMesh(core_axis_name="c", subcore_axis_name="s")

    @functools.partial(
        pl.kernel,
        mesh=mesh,
        out_type=jax.ShapeDtypeStruct((NC, N_PAD, D), jnp.float32),
        scratch_types=[
            pltpu.VMEM((GROUP, CHUNK), jnp.int32),
            pltpu.VMEM((GROUP, CHUNK), jnp.int32),
            pltpu.VMEM((CHUNK, D), jnp.float32),
            pltpu.VMEM((CHUNK, D), jnp.float32),
            pltpu.VMEM_SHARED((N_PAD, D), jnp.float32),
            pltpu.SemaphoreType.DMA,
            pltpu.SemaphoreType.DMA,
        ],
    )
    def sc_scatter(src_hbm, dst_hbm, x_hbm, out_hbm,
                   src_v, dst_v, rows_0, rows_1, acc_sh, sem_0, sem_1):
        c = lax.axis_index("c")
        s = lax.axis_index("s")
        w = c * NS + s

        pltpu.sync_copy(x_hbm.at[pl.ds(s * ROWS_PER_TILE, ROWS_PER_TILE)],
                        acc_sh.at[pl.ds(s * ROWS_PER_TILE, ROWS_PER_TILE)])
        plsc.subcore_barrier()

        rows = (rows_0, rows_1)
        sems = (sem_0, sem_1)
        for h in range(CHUNKS // GROUP):
            pltpu.sync_copy(src_hbm.at[w, pl.ds(h * GROUP, GROUP)], src_v)
            pltpu.sync_copy(dst_hbm.at[w, pl.ds(h * GROUP, GROUP)], dst_v)

            for b in range(2):
                pltpu.async_copy(x_hbm.at[src_v.at[b]], rows[b], sems[b])

            def body2(jj, carry):
                j0 = jj * 2
                for b in range(2):
                    j = j0 + b
                    pltpu.make_async_copy(x_hbm.at[src_v.at[j]], rows[b],
                                          sems[b]).wait()
                    pltpu.sync_copy(rows[b], acc_sh.at[dst_v.at[j]], add=True)

                    @pl.when(j + 2 < GROUP)
                    def _():
                        pltpu.async_copy(x_hbm.at[src_v.at[j + 2]], rows[b],
                                         sems[b])

                return carry

            lax.fori_loop(0, GROUP // 2, body2, 0)
        plsc.subcore_barrier()

        pltpu.sync_copy(acc_sh.at[pl.ds(s * ROWS_PER_TILE, ROWS_PER_TILE)],
                        out_hbm.at[c, pl.ds(s * ROWS_PER_TILE, ROWS_PER_TILE)])

    return sc_scatter


_sc_scatter = _sc_scatter_build()


def _mlp_mid_body(x_ref, p_ref, wa_ref, ba_ref, wb_ref, bb_ref, o_ref):
    t = p_ref[0] + p_ref[1] - x_ref[...]
    u = jnp.maximum(
        jnp.dot(t, wa_ref[...], preferred_element_type=jnp.float32)
        + ba_ref[...], 0.0)
    v = jnp.dot(u, wb_ref[...], preferred_element_type=jnp.float32) + bb_ref[...]
    o_ref[...] = jnp.maximum(v, 0.0)


def _mlp_last_body(x_ref, p_ref, wa_ref, ba_ref, wb_ref, bb_ref,
                   w3_ref, b3_ref, o_ref):
    t = p_ref[0] + p_ref[1] - x_ref[...]
    u = jnp.maximum(
        jnp.dot(t, wa_ref[...], preferred_element_type=jnp.float32)
        + ba_ref[...], 0.0)
    v = jnp.dot(u, wb_ref[...], preferred_element_type=jnp.float32) + bb_ref[...]
    h = jnp.maximum(v, 0.0)
    o_ref[...] = (jnp.dot(h, w3_ref[...], preferred_element_type=jnp.float32)
                  + b3_ref[...])


_RB = 1024


def _tc_mlp_mid(x, p, wa, ba, wb, bb):
    grid = (N_PAD // _RB,)
    return pl.pallas_call(
        _mlp_mid_body,
        grid=grid,
        in_specs=[
            pl.BlockSpec((_RB, D), lambda i: (i, 0)),
            pl.BlockSpec((NC, _RB, D), lambda i: (0, i, 0)),
            pl.BlockSpec((D, D), lambda i: (0, 0)),
            pl.BlockSpec((1, D), lambda i: (0, 0)),
            pl.BlockSpec((D, D), lambda i: (0, 0)),
            pl.BlockSpec((1, D), lambda i: (0, 0)),
        ],
        out_specs=pl.BlockSpec((_RB, D), lambda i: (i, 0)),
        out_shape=jax.ShapeDtypeStruct((N_PAD, D), jnp.float32),
    )(x, p, wa, ba, wb, bb)


def _tc_mlp_last(x, p, wa, ba, wb, bb, w3, b3):
    grid = (N // _RB + 1,)
    return pl.pallas_call(
        _mlp_last_body,
        grid=grid,
        in_specs=[
            pl.BlockSpec((_RB, D), lambda i: (i, 0)),
            pl.BlockSpec((NC, _RB, D), lambda i: (0, i, 0)),
            pl.BlockSpec((D, D), lambda i: (0, 0)),
            pl.BlockSpec((1, D), lambda i: (0, 0)),
            pl.BlockSpec((D, D), lambda i: (0, 0)),
            pl.BlockSpec((1, D), lambda i: (0, 0)),
            pl.BlockSpec((D, D_OUT), lambda i: (0, 0)),
            pl.BlockSpec((1, D_OUT), lambda i: (0, 0)),
        ],
        out_specs=pl.BlockSpec((_RB, D_OUT), lambda i: (i, 0)),
        out_shape=jax.ShapeDtypeStruct((N, D_OUT), jnp.float32),
    )(x, p, wa, ba, wb, bb, w3, b3)


def kernel(x, edge_index, W1a, b1a, W1b, b1b, W2a, b2a, W2b, b2b, W3, b3):
    src = edge_index[0].astype(jnp.int32)
    dst = edge_index[1].astype(jnp.int32)
    src_r = src.reshape(NW, CHUNKS, CHUNK)
    dst_r = dst.reshape(NW, CHUNKS, CHUNK)

    x_pad = jnp.concatenate([x, jnp.zeros((N_PAD - N, D), jnp.float32)])

    b1a2 = b1a.reshape(1, D)
    b1b2 = b1b.reshape(1, D)
    b2a2 = b2a.reshape(1, D)
    b2b2 = b2b.reshape(1, D)
    b32 = b3.reshape(1, D_OUT)

    p1 = _sc_scatter(src_r, dst_r, x_pad)
    h1 = _tc_mlp_mid(x_pad, p1, W1a, b1a2, W1b, b1b2)
    p2 = _sc_scatter(src_r, dst_r, h1)
    out = _tc_mlp_last(h1, p2, W2a, b2a2, W2b, b2b2, W3, b32)
    return out
